# Initial kernel scaffold; baseline (speedup 1.0000x reference)
import jax
import jax.numpy as jnp
from jax.experimental import pallas as pl


def kernel(x, xyz, idx, Wq, bq, Wk, bk, Wv, bv, Wo, bo, Wm1, bm1, Wm2, bm2):
    raise NotImplementedError("write your pallas kernel here")


# speedup vs baseline: 25.1761x; 25.1761x over previous
import functools
import math

import jax
import jax.numpy as jnp
from jax import lax
from jax.experimental import pallas as pl
from jax.experimental.pallas import tpu as pltpu
from jax.experimental.pallas import tpu_sc as plsc

XPAD = 128


def _build_table(x2, xyzp, idx2, Wk, bk, Wv, bv, N, BT):
    BNT, C = x2.shape
    K = idx2.shape[1]
    D = 2 * C + XPAD

    def body(x_ref, xyz_ref, idx_ref, wk_ref, bk_ref, wv_ref, bv_ref,
             tab_ref, idxa_ref):
        xb = x_ref[...]
        kf = jnp.dot(xb, wk_ref[...], preferred_element_type=jnp.float32) + bk_ref[...]
        vf = jnp.dot(xb, wv_ref[...], preferred_element_type=jnp.float32) + bv_ref[...]
        tab_ref[...] = jnp.concatenate([kf, vf, xyz_ref[...]], axis=1)
        off = (pl.program_id(0) * BT // N) * N
        idxa_ref[...] = idx_ref[...] + off

    grid = (BNT // BT,)
    return pl.pallas_call(
        body,
        grid=grid,
        in_specs=[
            pl.BlockSpec((BT, C), lambda i: (i, 0)),
            pl.BlockSpec((BT, XPAD), lambda i: (i, 0)),
            pl.BlockSpec((BT, K), lambda i: (i, 0)),
            pl.BlockSpec((C, C), lambda i: (0, 0)),
            pl.BlockSpec((1, C), lambda i: (0, 0)),
            pl.BlockSpec((C, C), lambda i: (0, 0)),
            pl.BlockSpec((1, C), lambda i: (0, 0)),
        ],
        out_specs=[
            pl.BlockSpec((BT, D), lambda i: (i, 0)),
            pl.BlockSpec((BT, K), lambda i: (i, 0)),
        ],
        out_shape=[
            jax.ShapeDtypeStruct((BNT, D), jnp.float32),
            jax.ShapeDtypeStruct((BNT, K), jnp.int32),
        ],
    )(x2, xyzp, idx2, Wk, bk, Wv, bv)


def _sc_gather(table, idx_flat):
    ROWS = idx_flat.shape[0]
    D = table.shape[1]
    NW = 32
    CH = 128
    RPW = ROWS // NW
    NCH = RPW // CH
    mesh = plsc.VectorSubcoreMesh(core_axis_name="c", subcore_axis_name="s")

    @functools.partial(
        pl.kernel,
        out_type=jax.ShapeDtypeStruct((ROWS, D), jnp.float32),
        mesh=mesh,
        scratch_types=[
            pltpu.VMEM((CH,), jnp.int32),
            pltpu.VMEM((CH, D), jnp.float32),
            pltpu.SemaphoreType.DMA,
        ],
    )
    def gather_k(tab_hbm, idx_hbm, out_hbm, idx_v, rows_v, sem):
        wid = lax.axis_index("s") * 2 + lax.axis_index("c")
        base = wid * RPW

        @pl.loop(0, NCH)
        def _(i):
            start = base + i * CH
            pltpu.sync_copy(idx_hbm.at[pl.ds(start, CH)], idx_v)
            pltpu.async_copy(tab_hbm.at[idx_v], rows_v, sem).wait()
            pltpu.sync_copy(rows_v, out_hbm.at[pl.ds(start, CH)])

    return gather_k(table, idx_flat)


def _attention(x, xyzp3, gath3, Wq, bq, Wm1p, bm1, Wm2, bm2, Wo, bo, BP):
    B, N, C = x.shape
    H = Wm2.shape[1]
    hd = C // H
    K = gath3.shape[1] // N
    D = gath3.shape[2]
    scale = 1.0 / math.sqrt(hd)

    def body(x_ref, xyz_ref, g_ref, wq_ref, bq_ref, wm1_ref, bm1_ref,
             wm2_ref, bm2_ref, wo_ref, bo_ref, o_ref):
        xb = x_ref[0]
        q = jnp.dot(xb, wq_ref[...], preferred_element_type=jnp.float32) + bq_ref[...]
        g = g_ref[0]
        kn = g[:, 0:C]
        vn = g[:, C:2 * C]
        nx = g[:, 2 * C:2 * C + XPAD]

        rel = xyz_ref[0][:, None, :] - nx.reshape(BP, K, XPAD)
        rel2 = rel.reshape(BP * K, XPAD)
        h1 = jnp.dot(rel2, wm1_ref[...], preferred_element_type=jnp.float32) + bm1_ref[...]
        h1 = jnp.maximum(h1, 0.0)
        hb = jnp.dot(h1, wm2_ref[...], preferred_element_type=jnp.float32) + bm2_ref[...]

        ce = lax.broadcasted_iota(jnp.int32, (C, H), 0) // hd
        he = lax.broadcasted_iota(jnp.int32, (C, H), 1)
        E = (ce == he).astype(jnp.float32)
        prod = (kn.reshape(BP, K, C) * q[:, None, :]).reshape(BP * K, C)
        s = jnp.dot(prod, E, preferred_element_type=jnp.float32) * scale + hb

        s3 = s.reshape(BP, K, H)
        m = jnp.max(s3, axis=1, keepdims=True)
        e = jnp.exp(s3 - m)
        den = jnp.sum(e, axis=1, keepdims=True)
        attn = (e / den).reshape(BP * K, H)

        aexp = jnp.dot(attn, E.T, preferred_element_type=jnp.float32)
        oa = (aexp * vn).reshape(BP, K, C).sum(axis=1)
        o_ref[0] = jnp.dot(oa, wo_ref[...], preferred_element_type=jnp.float32) + bo_ref[...]

    grid = (B, N // BP)
    return pl.pallas_call(
        body,
        grid=grid,
        in_specs=[
            pl.BlockSpec((1, BP, C), lambda b, i: (b, i, 0)),
            pl.BlockSpec((1, BP, XPAD), lambda b, i: (b, i, 0)),
            pl.BlockSpec((1, BP * K, D), lambda b, i: (b, i, 0)),
            pl.BlockSpec((C, C), lambda b, i: (0, 0)),
            pl.BlockSpec((1, C), lambda b, i: (0, 0)),
            pl.BlockSpec((XPAD, 32), lambda b, i: (0, 0)),
            pl.BlockSpec((1, 32), lambda b, i: (0, 0)),
            pl.BlockSpec((32, H), lambda b, i: (0, 0)),
            pl.BlockSpec((1, H), lambda b, i: (0, 0)),
            pl.BlockSpec((C, C), lambda b, i: (0, 0)),
            pl.BlockSpec((1, C), lambda b, i: (0, 0)),
        ],
        out_specs=pl.BlockSpec((1, BP, C), lambda b, i: (b, i, 0)),
        out_shape=jax.ShapeDtypeStruct((B, N, C), jnp.float32),
    )(x, xyzp3, gath3, Wq, bq, Wm1p, bm1, Wm2, bm2, Wo, bo)


def kernel(x, xyz, idx, Wq, bq, Wk, bk, Wv, bv, Wo, bo, Wm1, bm1, Wm2, bm2):
    B, N, C = x.shape
    K = idx.shape[2]
    H = Wm2.shape[1]

    x2 = x.reshape(B * N, C)
    xyzp = jnp.pad(xyz, ((0, 0), (0, 0), (0, XPAD - 3))).reshape(B * N, XPAD)
    idx2 = idx.reshape(B * N, K).astype(jnp.int32)
    Wm1p = jnp.pad(Wm1, ((0, XPAD - 3), (0, 0)))

    table, idxa = _build_table(x2, xyzp, idx2, Wk, bk.reshape(1, C),
                               Wv, bv.reshape(1, C), N, BT=256)
    gath = _sc_gather(table, idxa.reshape(B * N * K))
    gath3 = gath.reshape(B, N * K, table.shape[1])

    out = _attention(x, xyzp.reshape(B, N, XPAD), gath3,
                     Wq, bq.reshape(1, C), Wm1p, bm1.reshape(1, 32),
                     Wm2, bm2.reshape(1, H), Wo, bo.reshape(1, C), BP=128)
    return out


# speedup vs baseline: 26.9237x; 1.0694x over previous
import dataclasses
import functools
import math

import jax
import jax.numpy as jnp
from jax import lax
from jax.experimental import pallas as pl
from jax.experimental.pallas import tpu as pltpu
from jax.experimental.pallas import tpu_sc as plsc

XPAD = 4


def _pack_bf16_pair(lo_f32, hi_f32):
    lo = lax.convert_element_type(
        lax.bitcast_convert_type(lo_f32.astype(jnp.bfloat16), jnp.uint16), jnp.uint32)
    hi = lax.convert_element_type(
        lax.bitcast_convert_type(hi_f32.astype(jnp.bfloat16), jnp.uint16), jnp.uint32)
    return lax.bitcast_convert_type((hi << 16) | lo, jnp.int32)


def _unpack_bf16_pair(packed_i32):
    u = lax.bitcast_convert_type(packed_i32, jnp.uint32)
    lo = lax.bitcast_convert_type(
        lax.convert_element_type(u & jnp.uint32(0xFFFF), jnp.uint16), jnp.bfloat16)
    hi = lax.bitcast_convert_type(
        lax.convert_element_type(u >> 16, jnp.uint16), jnp.bfloat16)
    return lo.astype(jnp.float32), hi.astype(jnp.float32)


def _build_table(x2, idx2, Wk, bk, Wv, bv, N, BT):
    BNT, C = x2.shape
    K = idx2.shape[1]

    def body(x_ref, idx_ref, wk_ref, bk_ref, wv_ref, bv_ref, tab_ref, idxa_ref):
        xb = x_ref[...]
        kf = jnp.dot(xb, wk_ref[...], preferred_element_type=jnp.float32) + bk_ref[...]
        vf = jnp.dot(xb, wv_ref[...], preferred_element_type=jnp.float32) + bv_ref[...]
        tab_ref[...] = _pack_bf16_pair(kf, vf)
        off = (pl.program_id(0) * BT // N) * N
        idxa_ref[...] = idx_ref[...] + off

    grid = (BNT // BT,)
    return pl.pallas_call(
        body,
        grid=grid,
        in_specs=[
            pl.BlockSpec((BT, C), lambda i: (i, 0)),
            pl.BlockSpec((BT, K), lambda i: (i, 0)),
            pl.BlockSpec((C, C), lambda i: (0, 0)),
            pl.BlockSpec((1, C), lambda i: (0, 0)),
            pl.BlockSpec((C, C), lambda i: (0, 0)),
            pl.BlockSpec((1, C), lambda i: (0, 0)),
        ],
        out_specs=[
            pl.BlockSpec((BT, C), lambda i: (i, 0)),
            pl.BlockSpec((BT, K), lambda i: (i, 0)),
        ],
        out_shape=[
            jax.ShapeDtypeStruct((BNT, C), jnp.int32),
            jax.ShapeDtypeStruct((BNT, K), jnp.int32),
        ],
    )(x2, idx2, Wk, bk, Wv, bv)


def _sc_gather(table, idx_flat, xyzT, N, K):
    ROWS = idx_flat.shape[0]
    C = table.shape[1]
    NW = 32
    CH = 128
    L = 16
    RPW = ROWS // NW
    NCH = RPW // CH
    NKB = N * K
    mesh = plsc.VectorSubcoreMesh(core_axis_name="c", subcore_axis_name="s")
    cp = pltpu.CompilerParams()
    if "needs_layout_passes" in pltpu.CompilerParams.__dataclass_fields__:
        cp = dataclasses.replace(cp, needs_layout_passes=False)

    @functools.partial(
        pl.kernel,
        compiler_params=cp,
        out_type=[
            jax.ShapeDtypeStruct((ROWS, C), jnp.int32),
            jax.ShapeDtypeStruct((ROWS * XPAD,), jnp.float32),
        ],
        mesh=mesh,
        scratch_types=[
            pltpu.VMEM((N,), jnp.float32),
            pltpu.VMEM((N,), jnp.float32),
            pltpu.VMEM((N,), jnp.float32),
            pltpu.VMEM((CH,), jnp.int32),
            pltpu.VMEM((CH, C), jnp.int32),
            pltpu.VMEM((CH * XPAD,), jnp.float32),
            pltpu.SemaphoreType.DMA,
        ],
    )
    def gather_k(tab_hbm, idx_hbm, xyzT_hbm, out_hbm, nx_hbm,
                 cx_v, cy_v, cz_v, idx_v, rows_v, nx_v, sem):
        wid = lax.axis_index("s") * 2 + lax.axis_index("c")
        base = wid * RPW
        batch = base // NKB
        boff = batch * N

        pltpu.sync_copy(xyzT_hbm.at[pl.ds((batch * 3 + 0) * N, N)], cx_v)
        pltpu.sync_copy(xyzT_hbm.at[pl.ds((batch * 3 + 1) * N, N)], cy_v)
        pltpu.sync_copy(xyzT_hbm.at[pl.ds((batch * 3 + 2) * N, N)], cz_v)

        zeros = jnp.zeros((L,), jnp.float32)
        @pl.loop(0, CH * XPAD // L)
        def _(z):
            nx_v[pl.ds(z * L, L)] = zeros

        @pl.loop(0, NCH)
        def _(i):
            start = base + i * CH
            pltpu.sync_copy(idx_hbm.at[pl.ds(start, CH)], idx_v)
            cp = pltpu.async_copy(tab_hbm.at[idx_v], rows_v, sem)
            for j in range(CH // L):
                nb = idx_v[pl.ds(j * L, L)] - boff
                flat = (lax.iota(jnp.int32, L) + (j * L)) * XPAD
                gx = plsc.load_gather(cx_v, [nb])
                gy = plsc.load_gather(cy_v, [nb])
                gz = plsc.load_gather(cz_v, [nb])
                plsc.store_scatter(nx_v, [flat], gx)
                plsc.store_scatter(nx_v, [flat + 1], gy)
                plsc.store_scatter(nx_v, [flat + 2], gz)
            cp.wait()
            pltpu.sync_copy(rows_v, out_hbm.at[pl.ds(start, CH)])
            pltpu.sync_copy(nx_v, nx_hbm.at[pl.ds(start * XPAD, CH * XPAD)])

    return gather_k(table, idx_flat, xyzT)


def _attention(x, xyzp3, gath3, nx3, Wq, bq, Wm1p, bm1, Wm2, bm2, Wo, bo, BP):
    B, N, C = x.shape
    H = Wm2.shape[1]
    hd = C // H
    K = gath3.shape[1] // N
    scale = 1.0 / math.sqrt(hd)

    def body(x_ref, xyz_ref, g_ref, nx_ref, wq_ref, bq_ref, wm1_ref, bm1_ref,
             wm2_ref, bm2_ref, wo_ref, bo_ref, o_ref):
        xb = x_ref[0]
        q = jnp.dot(xb, wq_ref[...], preferred_element_type=jnp.float32) + bq_ref[...]
        kn, vn = _unpack_bf16_pair(g_ref[0])

        nx = nx_ref[0]
        rel = xyz_ref[0][:, None, :] - nx.reshape(BP, K, XPAD)
        rel2 = rel.reshape(BP * K, XPAD)
        h1 = jnp.dot(rel2, wm1_ref[...], preferred_element_type=jnp.float32) + bm1_ref[...]
        h1 = jnp.maximum(h1, 0.0)
        hb = jnp.dot(h1, wm2_ref[...], preferred_element_type=jnp.float32) + bm2_ref[...]

        ce = lax.broadcasted_iota(jnp.int32, (C, H), 0) // hd
        he = lax.broadcasted_iota(jnp.int32, (C, H), 1)
        E = (ce == he).astype(jnp.float32)
        prod = (kn.reshape(BP, K, C) * q[:, None, :]).reshape(BP * K, C)
        s = jnp.dot(prod, E, preferred_element_type=jnp.float32) * scale + hb

        s3 = s.reshape(BP, K, H)
        m = jnp.max(s3, axis=1, keepdims=True)
        e = jnp.exp(s3 - m)
        den = jnp.sum(e, axis=1, keepdims=True)
        attn = (e / den).reshape(BP * K, H)

        aexp = jnp.dot(attn, E.T, preferred_element_type=jnp.float32)
        oa = (aexp * vn).reshape(BP, K, C).sum(axis=1)
        o_ref[0] = jnp.dot(oa, wo_ref[...], preferred_element_type=jnp.float32) + bo_ref[...]

    grid = (B, N // BP)
    return pl.pallas_call(
        body,
        grid=grid,
        in_specs=[
            pl.BlockSpec((1, BP, C), lambda b, i: (b, i, 0)),
            pl.BlockSpec((1, BP, XPAD), lambda b, i: (b, i, 0)),
            pl.BlockSpec((1, BP * K, C), lambda b, i: (b, i, 0)),
            pl.BlockSpec((1, BP * K, XPAD), lambda b, i: (b, i, 0)),
            pl.BlockSpec((C, C), lambda b, i: (0, 0)),
            pl.BlockSpec((1, C), lambda b, i: (0, 0)),
            pl.BlockSpec((XPAD, 32), lambda b, i: (0, 0)),
            pl.BlockSpec((1, 32), lambda b, i: (0, 0)),
            pl.BlockSpec((32, H), lambda b, i: (0, 0)),
            pl.BlockSpec((1, H), lambda b, i: (0, 0)),
            pl.BlockSpec((C, C), lambda b, i: (0, 0)),
            pl.BlockSpec((1, C), lambda b, i: (0, 0)),
        ],
        out_specs=pl.BlockSpec((1, BP, C), lambda b, i: (b, i, 0)),
        out_shape=jax.ShapeDtypeStruct((B, N, C), jnp.float32),
    )(x, xyzp3, gath3, nx3, Wq, bq, Wm1p, bm1, Wm2, bm2, Wo, bo)


def kernel(x, xyz, idx, Wq, bq, Wk, bk, Wv, bv, Wo, bo, Wm1, bm1, Wm2, bm2):
    B, N, C = x.shape
    K = idx.shape[2]
    H = Wm2.shape[1]

    x2 = x.reshape(B * N, C)
    idx2 = idx.reshape(B * N, K).astype(jnp.int32)
    xyzT = jnp.transpose(xyz, (0, 2, 1)).reshape(B * 3 * N)
    xyzp = jnp.pad(xyz, ((0, 0), (0, 0), (0, XPAD - 3)))
    Wm1p = jnp.pad(Wm1, ((0, XPAD - 3), (0, 0)))

    table, idxa = _build_table(x2, idx2, Wk, bk.reshape(1, C),
                               Wv, bv.reshape(1, C), N, BT=256)
    gath, nx = _sc_gather(table, idxa.reshape(B * N * K), xyzT, N, K)
    gath3 = gath.reshape(B, N * K, C)
    nx3 = nx.reshape(B, N * K, XPAD)

    out = _attention(x, xyzp, gath3, nx3,
                     Wq, bq.reshape(1, C), Wm1p, bm1.reshape(1, 32),
                     Wm2, bm2.reshape(1, H), Wo, bo.reshape(1, C), BP=128)
    return out


# speedup vs baseline: 28.2233x; 1.0483x over previous
import dataclasses
import functools
import math

import jax
import jax.numpy as jnp
from jax import lax
from jax.experimental import pallas as pl
from jax.experimental.pallas import tpu as pltpu
from jax.experimental.pallas import tpu_sc as plsc

XPAD = 4


def _pack_bf16_pair(lo_f32, hi_f32):
    lo = lax.convert_element_type(
        lax.bitcast_convert_type(lo_f32.astype(jnp.bfloat16), jnp.uint16), jnp.uint32)
    hi = lax.convert_element_type(
        lax.bitcast_convert_type(hi_f32.astype(jnp.bfloat16), jnp.uint16), jnp.uint32)
    return lax.bitcast_convert_type((hi << 16) | lo, jnp.int32)


def _unpack_bf16_pair(packed_i32):
    u = lax.bitcast_convert_type(packed_i32, jnp.uint32)
    lo = lax.bitcast_convert_type(
        lax.convert_element_type(u & jnp.uint32(0xFFFF), jnp.uint16), jnp.bfloat16)
    hi = lax.bitcast_convert_type(
        lax.convert_element_type(u >> 16, jnp.uint16), jnp.bfloat16)
    return lo.astype(jnp.float32), hi.astype(jnp.float32)


def _build_table(x2, idx2, Wk, bk, Wv, bv, N, BT):
    BNT, C = x2.shape
    K = idx2.shape[1]

    def body(x_ref, idx_ref, wk_ref, bk_ref, wv_ref, bv_ref, tab_ref, idxa_ref):
        xb = x_ref[...]
        kf = jnp.dot(xb, wk_ref[...], preferred_element_type=jnp.float32) + bk_ref[...]
        vf = jnp.dot(xb, wv_ref[...], preferred_element_type=jnp.float32) + bv_ref[...]
        tab_ref[...] = _pack_bf16_pair(kf, vf)
        off = (pl.program_id(0) * BT // N) * N
        idxa_ref[...] = idx_ref[...] + off

    grid = (BNT // BT,)
    return pl.pallas_call(
        body,
        grid=grid,
        in_specs=[
            pl.BlockSpec((BT, C), lambda i: (i, 0)),
            pl.BlockSpec((BT, K), lambda i: (i, 0)),
            pl.BlockSpec((C, C), lambda i: (0, 0)),
            pl.BlockSpec((1, C), lambda i: (0, 0)),
            pl.BlockSpec((C, C), lambda i: (0, 0)),
            pl.BlockSpec((1, C), lambda i: (0, 0)),
        ],
        out_specs=[
            pl.BlockSpec((BT, C), lambda i: (i, 0)),
            pl.BlockSpec((BT, K), lambda i: (i, 0)),
        ],
        out_shape=[
            jax.ShapeDtypeStruct((BNT, C), jnp.int32),
            jax.ShapeDtypeStruct((BNT, K), jnp.int32),
        ],
    )(x2, idx2, Wk, bk, Wv, bv)


def _sc_gather(table, idx_flat, xyzT, N, K):
    ROWS = idx_flat.shape[0]
    C = table.shape[1]
    NW = 32
    CH = 128
    L = 16
    RPW = ROWS // NW
    NCH = RPW // CH
    NKB = N * K
    mesh = plsc.VectorSubcoreMesh(core_axis_name="c", subcore_axis_name="s")
    cp = pltpu.CompilerParams()
    if "needs_layout_passes" in pltpu.CompilerParams.__dataclass_fields__:
        cp = dataclasses.replace(cp, needs_layout_passes=False)

    @functools.partial(
        pl.kernel,
        compiler_params=cp,
        out_type=[
            jax.ShapeDtypeStruct((ROWS, C), jnp.int32),
            jax.ShapeDtypeStruct((ROWS * XPAD,), jnp.float32),
        ],
        mesh=mesh,
        scratch_types=[
            pltpu.VMEM((N,), jnp.float32),
            pltpu.VMEM((N,), jnp.float32),
            pltpu.VMEM((N,), jnp.float32),
            pltpu.VMEM((CH,), jnp.int32),
            pltpu.VMEM((CH, C), jnp.int32),
            pltpu.VMEM((CH * XPAD,), jnp.float32),
            pltpu.SemaphoreType.DMA,
        ],
    )
    def gather_k(tab_hbm, idx_hbm, xyzT_hbm, out_hbm, nx_hbm,
                 cx_v, cy_v, cz_v, idx_v, rows_v, nx_v, sem):
        wid = lax.axis_index("s") * 2 + lax.axis_index("c")
        base = wid * RPW
        batch = base // NKB
        boff = batch * N

        pltpu.sync_copy(xyzT_hbm.at[pl.ds((batch * 3 + 0) * N, N)], cx_v)
        pltpu.sync_copy(xyzT_hbm.at[pl.ds((batch * 3 + 1) * N, N)], cy_v)
        pltpu.sync_copy(xyzT_hbm.at[pl.ds((batch * 3 + 2) * N, N)], cz_v)

        zeros = jnp.zeros((L,), jnp.float32)
        @pl.loop(0, CH * XPAD // L)
        def _(z):
            nx_v[pl.ds(z * L, L)] = zeros

        @pl.loop(0, NCH)
        def _(i):
            start = base + i * CH
            pltpu.sync_copy(idx_hbm.at[pl.ds(start, CH)], idx_v)
            cp = pltpu.async_copy(tab_hbm.at[idx_v], rows_v, sem)
            for j in range(CH // L):
                nb = idx_v[pl.ds(j * L, L)] - boff
                flat = (lax.iota(jnp.int32, L) + (j * L)) * XPAD
                gx = plsc.load_gather(cx_v, [nb])
                gy = plsc.load_gather(cy_v, [nb])
                gz = plsc.load_gather(cz_v, [nb])
                plsc.store_scatter(nx_v, [flat], gx)
                plsc.store_scatter(nx_v, [flat + 1], gy)
                plsc.store_scatter(nx_v, [flat + 2], gz)
            cp.wait()
            pltpu.sync_copy(rows_v, out_hbm.at[pl.ds(start, CH)])
            pltpu.sync_copy(nx_v, nx_hbm.at[pl.ds(start * XPAD, CH * XPAD)])

    return gather_k(table, idx_flat, xyzT)


def _attention(x, xyzp3, gath3, nx3, Wq, bq, Wm1p, bm1, Wm2, bm2, Wo, bo, BP):
    B, N, C = x.shape
    H = Wm2.shape[1]
    hd = C // H
    K = gath3.shape[1] // N
    scale = 1.0 / math.sqrt(hd)

    def body(x_ref, xyz_ref, g_ref, nx_ref, wq_ref, bq_ref, wm1_ref, bm1_ref,
             wm2_ref, bm2_ref, wo_ref, bo_ref, o_ref):
        bf = jnp.bfloat16
        xb = x_ref[0]
        q = jnp.dot(xb, wq_ref[...], preferred_element_type=jnp.float32) + bq_ref[...]
        q_bf = q.astype(bf)
        u = lax.bitcast_convert_type(g_ref[0], jnp.uint32)
        kn_bf = lax.bitcast_convert_type(
            lax.convert_element_type(u & jnp.uint32(0xFFFF), jnp.uint16), bf)
        vn_bf = lax.bitcast_convert_type(
            lax.convert_element_type(u >> 16, jnp.uint16), bf)

        nx = nx_ref[0]
        rel = xyz_ref[0][:, None, :] - nx.reshape(BP, K, XPAD)
        rel_bf = rel.reshape(BP * K, XPAD).astype(bf)
        h1 = jnp.dot(rel_bf, wm1_ref[...].astype(bf),
                     preferred_element_type=jnp.float32) + bm1_ref[...]
        h1_bf = jnp.maximum(h1, 0.0).astype(bf)
        hb = jnp.dot(h1_bf, wm2_ref[...].astype(bf),
                     preferred_element_type=jnp.float32) + bm2_ref[...]

        ce = lax.broadcasted_iota(jnp.int32, (C, H), 0) // hd
        he = lax.broadcasted_iota(jnp.int32, (C, H), 1)
        E_bf = (ce == he).astype(bf)
        prod = (kn_bf.reshape(BP, K, C) * q_bf[:, None, :]).reshape(BP * K, C)
        s = jnp.dot(prod, E_bf, preferred_element_type=jnp.float32) * scale + hb

        s3 = s.reshape(BP, K, H)
        m = jnp.max(s3, axis=1, keepdims=True)
        e = jnp.exp(s3 - m)
        den = jnp.sum(e, axis=1, keepdims=True)
        attn_bf = (e / den).reshape(BP * K, H).astype(bf)

        aexp = jnp.dot(attn_bf, E_bf.T, preferred_element_type=jnp.float32).astype(bf)
        oa = (aexp * vn_bf).reshape(BP, K, C).sum(axis=1).astype(jnp.float32)
        o_ref[0] = jnp.dot(oa, wo_ref[...], preferred_element_type=jnp.float32) + bo_ref[...]

    grid = (B, N // BP)
    return pl.pallas_call(
        body,
        grid=grid,
        in_specs=[
            pl.BlockSpec((1, BP, C), lambda b, i: (b, i, 0)),
            pl.BlockSpec((1, BP, XPAD), lambda b, i: (b, i, 0)),
            pl.BlockSpec((1, BP * K, C), lambda b, i: (b, i, 0)),
            pl.BlockSpec((1, BP * K, XPAD), lambda b, i: (b, i, 0)),
            pl.BlockSpec((C, C), lambda b, i: (0, 0)),
            pl.BlockSpec((1, C), lambda b, i: (0, 0)),
            pl.BlockSpec((XPAD, 32), lambda b, i: (0, 0)),
            pl.BlockSpec((1, 32), lambda b, i: (0, 0)),
            pl.BlockSpec((32, H), lambda b, i: (0, 0)),
            pl.BlockSpec((1, H), lambda b, i: (0, 0)),
            pl.BlockSpec((C, C), lambda b, i: (0, 0)),
            pl.BlockSpec((1, C), lambda b, i: (0, 0)),
        ],
        out_specs=pl.BlockSpec((1, BP, C), lambda b, i: (b, i, 0)),
        out_shape=jax.ShapeDtypeStruct((B, N, C), jnp.float32),
    )(x, xyzp3, gath3, nx3, Wq, bq, Wm1p, bm1, Wm2, bm2, Wo, bo)


def kernel(x, xyz, idx, Wq, bq, Wk, bk, Wv, bv, Wo, bo, Wm1, bm1, Wm2, bm2):
    B, N, C = x.shape
    K = idx.shape[2]
    H = Wm2.shape[1]

    x2 = x.reshape(B * N, C)
    idx2 = idx.reshape(B * N, K).astype(jnp.int32)
    xyzT = jnp.transpose(xyz, (0, 2, 1)).reshape(B * 3 * N)
    xyzp = jnp.pad(xyz, ((0, 0), (0, 0), (0, XPAD - 3)))
    Wm1p = jnp.pad(Wm1, ((0, XPAD - 3), (0, 0)))

    table, idxa = _build_table(x2, idx2, Wk, bk.reshape(1, C),
                               Wv, bv.reshape(1, C), N, BT=256)
    gath, nx = _sc_gather(table, idxa.reshape(B * N * K), xyzT, N, K)
    gath3 = gath.reshape(B, N * K, C)
    nx3 = nx.reshape(B, N * K, XPAD)

    out = _attention(x, xyzp, gath3, nx3,
                     Wq, bq.reshape(1, C), Wm1p, bm1.reshape(1, 32),
                     Wm2, bm2.reshape(1, H), Wo, bo.reshape(1, C), BP=256)
    return out


# speedup vs baseline: 35.2436x; 1.2487x over previous
import dataclasses
import functools
import math

import jax
import jax.numpy as jnp
from jax import lax
from jax.experimental import pallas as pl
from jax.experimental.pallas import tpu as pltpu
from jax.experimental.pallas import tpu_sc as plsc

XPAD = 4


def _pack_bf16_pair(lo_f32, hi_f32):
    lo = lax.convert_element_type(
        lax.bitcast_convert_type(lo_f32.astype(jnp.bfloat16), jnp.uint16), jnp.uint32)
    hi = lax.convert_element_type(
        lax.bitcast_convert_type(hi_f32.astype(jnp.bfloat16), jnp.uint16), jnp.uint32)
    return lax.bitcast_convert_type((hi << 16) | lo, jnp.int32)


def _unpack_bf16_pair(packed_i32):
    u = lax.bitcast_convert_type(packed_i32, jnp.uint32)
    lo = lax.bitcast_convert_type(
        lax.convert_element_type(u & jnp.uint32(0xFFFF), jnp.uint16), jnp.bfloat16)
    hi = lax.bitcast_convert_type(
        lax.convert_element_type(u >> 16, jnp.uint16), jnp.bfloat16)
    return lo.astype(jnp.float32), hi.astype(jnp.float32)


def _build_table(x2, idx2, Wk, bk, Wv, bv, N, BT):
    BNT, C = x2.shape
    K = idx2.shape[1]

    def body(x_ref, idx_ref, wk_ref, bk_ref, wv_ref, bv_ref, tab_ref, idxa_ref):
        xb = x_ref[...].astype(jnp.bfloat16)
        kf = jnp.dot(xb, wk_ref[...].astype(jnp.bfloat16),
                     preferred_element_type=jnp.float32) + bk_ref[...]
        vf = jnp.dot(xb, wv_ref[...].astype(jnp.bfloat16),
                     preferred_element_type=jnp.float32) + bv_ref[...]
        tab_ref[...] = _pack_bf16_pair(kf, vf)
        off = (pl.program_id(0) * BT // N) * N
        idxa_ref[...] = idx_ref[...] + off

    grid = (BNT // BT,)
    return pl.pallas_call(
        body,
        grid=grid,
        in_specs=[
            pl.BlockSpec((BT, C), lambda i: (i, 0)),
            pl.BlockSpec((BT, K), lambda i: (i, 0)),
            pl.BlockSpec((C, C), lambda i: (0, 0)),
            pl.BlockSpec((1, C), lambda i: (0, 0)),
            pl.BlockSpec((C, C), lambda i: (0, 0)),
            pl.BlockSpec((1, C), lambda i: (0, 0)),
        ],
        out_specs=[
            pl.BlockSpec((BT, C), lambda i: (i, 0)),
            pl.BlockSpec((BT, K), lambda i: (i, 0)),
        ],
        out_shape=[
            jax.ShapeDtypeStruct((BNT, C), jnp.int32),
            jax.ShapeDtypeStruct((BNT, K), jnp.int32),
        ],
    )(x2, idx2, Wk, bk, Wv, bv)


def _sc_gather(table, idx_flat, xyzT, N, K):
    ROWS = idx_flat.shape[0]
    C = table.shape[1]
    NW = 32
    CH = 128
    L = 16
    RPW = ROWS // NW
    NCH = RPW // CH
    NKB = N * K
    mesh = plsc.VectorSubcoreMesh(core_axis_name="c", subcore_axis_name="s")
    cp = pltpu.CompilerParams()
    if "needs_layout_passes" in pltpu.CompilerParams.__dataclass_fields__:
        cp = dataclasses.replace(cp, needs_layout_passes=False)

    B = ROWS // NKB

    @functools.partial(
        pl.kernel,
        compiler_params=cp,
        out_type=[
            jax.ShapeDtypeStruct((B, NKB, C), jnp.int32),
            jax.ShapeDtypeStruct((B, NKB, XPAD), jnp.float32),
        ],
        mesh=mesh,
        scratch_types=[
            pltpu.VMEM((N,), jnp.float32),
            pltpu.VMEM((N,), jnp.float32),
            pltpu.VMEM((N,), jnp.float32),
            pltpu.VMEM((CH,), jnp.int32),
            pltpu.VMEM((CH, C), jnp.int32),
            pltpu.VMEM((CH, XPAD), jnp.float32),
            pltpu.SemaphoreType.DMA,
        ],
    )
    def gather_k(tab_hbm, idx_hbm, xyzT_hbm, out_hbm, nx_hbm,
                 cx_v, cy_v, cz_v, idx_v, rows_v, nx_v, sem):
        wid = lax.axis_index("s") * 2 + lax.axis_index("c")
        base = wid * RPW
        batch = base // NKB
        boff = batch * N

        pltpu.sync_copy(xyzT_hbm.at[pl.ds((batch * 3 + 0) * N, N)], cx_v)
        pltpu.sync_copy(xyzT_hbm.at[pl.ds((batch * 3 + 1) * N, N)], cy_v)
        pltpu.sync_copy(xyzT_hbm.at[pl.ds((batch * 3 + 2) * N, N)], cz_v)

        zeros = jnp.zeros((L,), jnp.float32)

        @pl.loop(0, NCH)
        def _(i):
            start = base + i * CH
            lstart = start - batch * NKB
            pltpu.sync_copy(idx_hbm.at[pl.ds(start, CH)], idx_v)
            cp = pltpu.async_copy(tab_hbm.at[idx_v], rows_v, sem)
            for j in range(CH // L):
                nb = idx_v[pl.ds(j * L, L)] - boff
                rows16 = lax.iota(jnp.int32, L) + (j * L)
                gx = plsc.load_gather(cx_v, [nb])
                gy = plsc.load_gather(cy_v, [nb])
                gz = plsc.load_gather(cz_v, [nb])
                plsc.store_scatter(nx_v, [rows16, jnp.full((L,), 0, jnp.int32)], gx)
                plsc.store_scatter(nx_v, [rows16, jnp.full((L,), 1, jnp.int32)], gy)
                plsc.store_scatter(nx_v, [rows16, jnp.full((L,), 2, jnp.int32)], gz)
                plsc.store_scatter(nx_v, [rows16, jnp.full((L,), 3, jnp.int32)], zeros)
            cp.wait()
            pltpu.sync_copy(rows_v, out_hbm.at[batch, pl.ds(lstart, CH)])
            pltpu.sync_copy(nx_v, nx_hbm.at[batch, pl.ds(lstart, CH)])

    return gather_k(table, idx_flat, xyzT)


def _attention(x, xyzp3, gath3, nx3, Wq, bq, Wm1p, bm1, Wm2r, bm2r, Wo, bo,
               hd, BP):
    B, N, C = x.shape
    K = gath3.shape[1] // N
    scale = 1.0 / math.sqrt(hd)

    def body(x_ref, xyz_ref, g_ref, nx_ref, wq_ref, bq_ref, wm1_ref, bm1_ref,
             wm2_ref, bm2_ref, wo_ref, bo_ref, o_ref):
        bf = jnp.bfloat16
        xb = x_ref[0]
        q = jnp.dot(xb, wq_ref[...], preferred_element_type=jnp.float32) + bq_ref[...]
        q_bf = q.astype(bf)
        u = lax.bitcast_convert_type(g_ref[0], jnp.uint32)
        kn_bf = lax.bitcast_convert_type(
            lax.convert_element_type(u & jnp.uint32(0xFFFF), jnp.uint16), bf)
        vn_bf = lax.bitcast_convert_type(
            lax.convert_element_type(u >> 16, jnp.uint16), bf)

        nx = nx_ref[0]
        rel = xyz_ref[0][:, None, :] - nx.reshape(BP, K, XPAD)
        rel_bf = rel.reshape(BP * K, XPAD).astype(bf)
        h1 = jnp.dot(rel_bf, wm1_ref[...].astype(bf),
                     preferred_element_type=jnp.float32) + bm1_ref[...]
        h1_bf = jnp.maximum(h1, 0.0).astype(bf)
        hb = jnp.dot(h1_bf, wm2_ref[...].astype(bf),
                     preferred_element_type=jnp.float32) + bm2_ref[...]

        ce = lax.broadcasted_iota(jnp.int32, (C, C), 0) // hd
        je = lax.broadcasted_iota(jnp.int32, (C, C), 1) // hd
        E2 = jnp.where(ce == je, scale, 0.0).astype(bf)
        prod = (kn_bf.reshape(BP, K, C) * q_bf[:, None, :]).reshape(BP * K, C)
        s = jnp.dot(prod, E2, preferred_element_type=jnp.float32) + hb

        s3 = s.reshape(BP, K, C)
        m = jnp.max(s3, axis=1, keepdims=True)
        e = jnp.exp(s3 - m)
        den = jnp.sum(e, axis=1, keepdims=True)
        attn_bf = (e / den).astype(bf)

        oa = (attn_bf * vn_bf.reshape(BP, K, C)).sum(axis=1).astype(jnp.float32)
        o_ref[0] = jnp.dot(oa, wo_ref[...], preferred_element_type=jnp.float32) + bo_ref[...]

    grid = (B, N // BP)
    return pl.pallas_call(
        body,
        grid=grid,
        in_specs=[
            pl.BlockSpec((1, BP, C), lambda b, i: (b, i, 0)),
            pl.BlockSpec((1, BP, XPAD), lambda b, i: (b, i, 0)),
            pl.BlockSpec((1, BP * K, C), lambda b, i: (b, i, 0)),
            pl.BlockSpec((1, BP * K, XPAD), lambda b, i: (b, i, 0)),
            pl.BlockSpec((C, C), lambda b, i: (0, 0)),
            pl.BlockSpec((1, C), lambda b, i: (0, 0)),
            pl.BlockSpec((XPAD, 32), lambda b, i: (0, 0)),
            pl.BlockSpec((1, 32), lambda b, i: (0, 0)),
            pl.BlockSpec((32, C), lambda b, i: (0, 0)),
            pl.BlockSpec((1, C), lambda b, i: (0, 0)),
            pl.BlockSpec((C, C), lambda b, i: (0, 0)),
            pl.BlockSpec((1, C), lambda b, i: (0, 0)),
        ],
        out_specs=pl.BlockSpec((1, BP, C), lambda b, i: (b, i, 0)),
        out_shape=jax.ShapeDtypeStruct((B, N, C), jnp.float32),
    )(x, xyzp3, gath3, nx3, Wq, bq, Wm1p, bm1, Wm2r, bm2r, Wo, bo)


def kernel(x, xyz, idx, Wq, bq, Wk, bk, Wv, bv, Wo, bo, Wm1, bm1, Wm2, bm2):
    B, N, C = x.shape
    K = idx.shape[2]
    H = Wm2.shape[1]

    x2 = x.reshape(B * N, C)
    idx2 = idx.reshape(B * N, K).astype(jnp.int32)
    xyzT = jnp.transpose(xyz, (0, 2, 1)).reshape(B * 3 * N)
    xyzp = jnp.pad(xyz, ((0, 0), (0, 0), (0, XPAD - 3)))
    Wm1p = jnp.pad(Wm1, ((0, XPAD - 3), (0, 0)))

    hd = C // H
    Wm2r = jnp.repeat(Wm2, hd, axis=1)
    bm2r = jnp.repeat(bm2.reshape(1, H), hd, axis=1)

    table, idxa = _build_table(x2, idx2, Wk, bk.reshape(1, C),
                               Wv, bv.reshape(1, C), N, BT=256)
    gath3, nx3 = _sc_gather(table, idxa.reshape(B * N * K), xyzT, N, K)

    out = _attention(x, xyzp, gath3, nx3,
                     Wq, bq.reshape(1, C), Wm1p, bm1.reshape(1, 32),
                     Wm2r, bm2r, Wo, bo.reshape(1, C), hd=hd, BP=256)
    return out
